# Initial kernel scaffold; baseline (speedup 1.0000x reference)
import jax
import jax.numpy as jnp
from jax.experimental import pallas as pl


def kernel(x, edge_index, W_lin, b_lin, W_attn, b_attn):
    raise NotImplementedError("write your pallas kernel here")


# speedup vs baseline: 29.3136x; 29.3136x over previous
import functools

import jax
import jax.numpy as jnp
from jax import lax
from jax.experimental import pallas as pl
from jax.experimental.pallas import tpu as pltpu
from jax.experimental.pallas import tpu_sc as plsc

N = 10000
E = 320000
IN_DIM = 128
OPH = 32

NC = 2
NS = 16
NW = NC * NS
CHUNK = 128
NCHUNK = 80
EPW = NCHUNK * CHUNK
E_PAD = NW * EPW


def _pre_body(x_ref, wlt_ref, bl_ref, wab_ref, bab_ref, h_ref, ab_ref):
    h = jnp.dot(x_ref[...], wlt_ref[...], preferred_element_type=jnp.float32)
    h = h + bl_ref[...]
    h_ref[...] = h
    ab_ref[...] = (
        jnp.dot(h, wab_ref[...], preferred_element_type=jnp.float32)
        + bab_ref[...]
    )


def _pre(x, wlt, bl, wab, bab):
    return pl.pallas_call(
        _pre_body,
        out_shape=(
            jax.ShapeDtypeStruct((N, OPH), jnp.float32),
            jax.ShapeDtypeStruct((N, 8), jnp.float32),
        ),
    )(x, wlt, bl, wab, bab)


def _sc_body(h_hbm, alpha_hbm, beta_hbm, s_hbm, r_hbm, z32_hbm, z1_hbm,
             outp_hbm, denp_hbm,
             alpha_t, beta_t, sbuf, rbuf, rows, pfull, out_acc, den_acc,
             sem0, sem1):
    cid = lax.axis_index("c")
    sid = lax.axis_index("s")
    wid = sid * NC + cid

    if True:
        pltpu.sync_copy(alpha_hbm, alpha_t)
        pltpu.sync_copy(beta_hbm, beta_t)
        pltpu.sync_copy(s_hbm.at[wid], sbuf)
        pltpu.sync_copy(r_hbm.at[wid], rbuf)

        @pl.when(sid == 0)
        def _():
            pltpu.sync_copy(z32_hbm, out_acc)
            pltpu.sync_copy(z1_hbm, den_acc)

        ebase = wid * EPW
        lane = lax.iota(jnp.int32, 16)

        def _pchunk(c, carry):
            for j in range(CHUNK // 16):
                sv = sbuf[c, pl.ds(j * 16, 16)]
                rv = rbuf[c, pl.ds(j * 16, 16)]
                a = plsc.load_gather(alpha_t, [sv])
                b = plsc.load_gather(beta_t, [rv])
                l = a + b
                l = jnp.where(l >= 0.0, l, l * jnp.float32(0.01))
                p = jnp.exp(l)
                gid = ebase + c * CHUNK + j * 16 + lane
                p = jnp.where(gid < E, p, jnp.float32(0.0))
                pfull[c, pl.ds(j * 16, 16)] = p
            return carry

        lax.fori_loop(0, NCHUNK, _pchunk, 0)
        plsc.subcore_barrier()

        def _start(c, rbank, sem):
            pltpu.async_copy(h_hbm.at[sbuf.at[c]], rbank, sem)

        _start(0, rows.at[0], sem0)
        _start(1, rows.at[1], sem1)

        def _process(c, buf, sem):
            rbank = rows.at[buf]
            pltpu.make_async_copy(h_hbm.at[sbuf.at[c]], rbank, sem).wait()
            cvec = jnp.broadcast_to(c, (16,)).astype(jnp.int32)
            for e in range(CHUNK):
                evec = jnp.full((16,), e, jnp.int32)
                pe = plsc.load_gather(pfull, [cvec, evec])
                rows[buf, e, pl.ds(0, 16)] = rows[buf, e, pl.ds(0, 16)] * pe
                rows[buf, e, pl.ds(16, 16)] = rows[buf, e, pl.ds(16, 16)] * pe
            pltpu.sync_copy(rbank, out_acc.at[rbuf.at[c]], add=True)
            pltpu.sync_copy(pfull.at[c], den_acc.at[rbuf.at[c]], add=True)

            @pl.when(c + 2 < NCHUNK)
            def _():
                _start(c + 2, rbank, sem)

        def _pair(t, carry):
            _process(2 * t, 0, sem0)
            _process(2 * t + 1, 1, sem1)
            return carry

        lax.fori_loop(0, NCHUNK // 2, _pair, 0)
        plsc.subcore_barrier()

        @pl.when(sid == 0)
        def _():
            pltpu.sync_copy(out_acc, outp_hbm.at[cid])
            pltpu.sync_copy(den_acc, denp_hbm.at[cid, 0])

def _sc_edge_pass(h, alpha, beta, s_p, r_p, z32, z1):
    mesh = plsc.VectorSubcoreMesh(core_axis_name="c", subcore_axis_name="s")
    return pl.kernel(
        _sc_body,
        out_type=(
            jax.ShapeDtypeStruct((NC, N, OPH), jnp.float32),
            jax.ShapeDtypeStruct((NC, 1, N), jnp.float32),
        ),
        mesh=mesh,
        compiler_params=pltpu.CompilerParams(
            needs_layout_passes=False, use_tc_tiling_on_sc=False),
        scratch_types=[
            pltpu.VMEM((N,), jnp.float32),
            pltpu.VMEM((N,), jnp.float32),
            pltpu.VMEM((NCHUNK, CHUNK), jnp.int32),
            pltpu.VMEM((NCHUNK, CHUNK), jnp.int32),
            pltpu.VMEM((2, CHUNK, OPH), jnp.float32),
            pltpu.VMEM((NCHUNK, CHUNK), jnp.float32),
            pltpu.VMEM_SHARED((N, OPH), jnp.float32),
            pltpu.VMEM_SHARED((N,), jnp.float32),
            pltpu.SemaphoreType.DMA,
            pltpu.SemaphoreType.DMA,
        ],
    )(h, alpha, beta, s_p, r_p, z32, z1)


def _post_body(op_ref, dp_ref, o_ref):
    num = op_ref[0] + op_ref[1]
    den = dp_ref[0, 0] + dp_ref[1, 0]
    den = den[:, None]
    o_ref[...] = jnp.where(den > 0.0, num / den, jnp.float32(0.0))


def _post(outp, denp):
    return pl.pallas_call(
        _post_body,
        out_shape=jax.ShapeDtypeStruct((N, OPH), jnp.float32),
    )(outp, denp)


@jax.jit
def kernel(x, edge_index, W_lin, b_lin, W_attn, b_attn):
    wlt = W_lin.T
    bl = b_lin.reshape(1, OPH)
    wa = W_attn[0]
    wab = jnp.zeros((OPH, 8), jnp.float32)
    wab = wab.at[:, 0].set(wa[:OPH]).at[:, 1].set(wa[OPH:])
    bab = jnp.zeros((1, 8), jnp.float32).at[0, 1].set(b_attn[0])

    h, ab = _pre(x, wlt, bl, wab, bab)
    alpha = ab[:, 0]
    beta = ab[:, 1]

    pad = E_PAD - E
    s_p = jnp.pad(edge_index[0], (0, pad)).reshape(NW, NCHUNK, CHUNK)
    r_p = jnp.pad(edge_index[1], (0, pad)).reshape(NW, NCHUNK, CHUNK)
    z32 = jnp.zeros((N, OPH), jnp.float32)
    z1 = jnp.zeros((N,), jnp.float32)

    outp, denp = _sc_edge_pass(h, alpha, beta, s_p, r_p, z32, z1)
    return _post(outp, denp)


# speedup vs baseline: 30.4682x; 1.0394x over previous
import functools

import jax
import jax.numpy as jnp
from jax import lax
from jax.experimental import pallas as pl
from jax.experimental.pallas import tpu as pltpu
from jax.experimental.pallas import tpu_sc as plsc

N = 10000
E = 320000
IN_DIM = 128
OPH = 32

NC = 2
NS = 16
NW = NC * NS
CHUNK = 128
NCHUNK = 80
EPW = NCHUNK * CHUNK
E_PAD = NW * EPW


def _pre_body(x_ref, wlt_ref, bl_ref, wab_ref, bab_ref, h_ref, ab_ref):
    h = jnp.dot(x_ref[...], wlt_ref[...], preferred_element_type=jnp.float32)
    h = h + bl_ref[...]
    h_ref[...] = h
    ab_ref[...] = (
        jnp.dot(h, wab_ref[...], preferred_element_type=jnp.float32)
        + bab_ref[...]
    )


def _pre(x, wlt, bl, wab, bab):
    return pl.pallas_call(
        _pre_body,
        out_shape=(
            jax.ShapeDtypeStruct((N, OPH), jnp.float32),
            jax.ShapeDtypeStruct((N, 8), jnp.float32),
        ),
    )(x, wlt, bl, wab, bab)


def _sc_body(h_hbm, alpha_hbm, beta_hbm, s_hbm, r_hbm, z32_hbm, z1_hbm,
             outp_hbm, denp_hbm,
             alpha_t, beta_t, sbuf, rbuf, rows, scat, pfull, out_acc, den_acc,
             gsem, ssem, dsem):
    cid = lax.axis_index("c")
    sid = lax.axis_index("s")
    wid = sid * NC + cid

    if True:
        pltpu.sync_copy(alpha_hbm, alpha_t)
        pltpu.sync_copy(beta_hbm, beta_t)
        pltpu.sync_copy(s_hbm.at[wid], sbuf)
        pltpu.sync_copy(r_hbm.at[wid], rbuf)

        @pl.when(sid == 0)
        def _():
            pltpu.sync_copy(z32_hbm, out_acc)
            pltpu.sync_copy(z1_hbm, den_acc)

        ebase = wid * EPW
        lane = lax.iota(jnp.int32, 16)

        def _pchunk(c, carry):
            for j in range(CHUNK // 16):
                sv = sbuf[c, pl.ds(j * 16, 16)]
                rv = rbuf[c, pl.ds(j * 16, 16)]
                a = plsc.load_gather(alpha_t, [sv])
                b = plsc.load_gather(beta_t, [rv])
                l = a + b
                l = jnp.where(l >= 0.0, l, l * jnp.float32(0.01))
                p = jnp.exp(l)
                gid = ebase + c * CHUNK + j * 16 + lane
                p = jnp.where(gid < E, p, jnp.float32(0.0))
                pfull[c, pl.ds(j * 16, 16)] = p
            return carry

        lax.fori_loop(0, NCHUNK, _pchunk, 0)
        plsc.subcore_barrier()

        def _gstart(c, buf):
            pltpu.async_copy(h_hbm.at[sbuf.at[c]], rows.at[buf], gsem.at[buf])

        def _gwait(c, buf):
            pltpu.make_async_copy(
                h_hbm.at[sbuf.at[c]], rows.at[buf], gsem.at[buf]).wait()

        def _sstart(c, buf):
            pltpu.async_copy(scat.at[buf], out_acc.at[rbuf.at[c]],
                             ssem.at[buf], add=True)
            pltpu.async_copy(pfull.at[c], den_acc.at[rbuf.at[c]],
                             dsem.at[buf], add=True)

        def _swait(c, buf):
            pltpu.make_async_copy(scat.at[buf], out_acc.at[rbuf.at[c]],
                                  ssem.at[buf]).wait()
            pltpu.make_async_copy(pfull.at[c], den_acc.at[rbuf.at[c]],
                                  dsem.at[buf]).wait()

        _gstart(0, 0)
        _gstart(1, 1)

        def _process(c, buf):
            _gwait(c, buf)

            @pl.when(c >= 2)
            def _():
                _swait(c - 2, buf)

            cvec = jnp.broadcast_to(c, (16,)).astype(jnp.int32)
            for e in range(CHUNK):
                evec = jnp.full((16,), e, jnp.int32)
                pe = plsc.load_gather(pfull, [cvec, evec])
                scat[buf, e, pl.ds(0, 16)] = rows[buf, e, pl.ds(0, 16)] * pe
                scat[buf, e, pl.ds(16, 16)] = rows[buf, e, pl.ds(16, 16)] * pe
            _sstart(c, buf)

            @pl.when(c + 2 < NCHUNK)
            def _():
                _gstart(c + 2, buf)

        def _pair(t, carry):
            _process(2 * t, 0)
            _process(2 * t + 1, 1)
            return carry

        lax.fori_loop(0, NCHUNK // 2, _pair, 0)
        _swait(NCHUNK - 2, 0)
        _swait(NCHUNK - 1, 1)
        plsc.subcore_barrier()

        @pl.when(sid == 0)
        def _():
            pltpu.sync_copy(out_acc, outp_hbm.at[cid])
            pltpu.sync_copy(den_acc, denp_hbm.at[cid, 0])

def _sc_edge_pass(h, alpha, beta, s_p, r_p, z32, z1):
    mesh = plsc.VectorSubcoreMesh(core_axis_name="c", subcore_axis_name="s")
    return pl.kernel(
        _sc_body,
        out_type=(
            jax.ShapeDtypeStruct((NC, N, OPH), jnp.float32),
            jax.ShapeDtypeStruct((NC, 1, N), jnp.float32),
        ),
        mesh=mesh,
        compiler_params=pltpu.CompilerParams(
            needs_layout_passes=False, use_tc_tiling_on_sc=False),
        scratch_types=[
            pltpu.VMEM((N,), jnp.float32),
            pltpu.VMEM((N,), jnp.float32),
            pltpu.VMEM((NCHUNK, CHUNK), jnp.int32),
            pltpu.VMEM((NCHUNK, CHUNK), jnp.int32),
            pltpu.VMEM((2, CHUNK, OPH), jnp.float32),
            pltpu.VMEM((2, CHUNK, OPH), jnp.float32),
            pltpu.VMEM((NCHUNK, CHUNK), jnp.float32),
            pltpu.VMEM_SHARED((N, OPH), jnp.float32),
            pltpu.VMEM_SHARED((N,), jnp.float32),
            pltpu.SemaphoreType.DMA((2,)),
            pltpu.SemaphoreType.DMA((2,)),
            pltpu.SemaphoreType.DMA((2,)),
        ],
    )(h, alpha, beta, s_p, r_p, z32, z1)


def _post_body(op_ref, dp_ref, o_ref):
    num = op_ref[0] + op_ref[1]
    den = dp_ref[0, 0] + dp_ref[1, 0]
    den = den[:, None]
    o_ref[...] = jnp.where(den > 0.0, num / den, jnp.float32(0.0))


def _post(outp, denp):
    return pl.pallas_call(
        _post_body,
        out_shape=jax.ShapeDtypeStruct((N, OPH), jnp.float32),
    )(outp, denp)


@jax.jit
def kernel(x, edge_index, W_lin, b_lin, W_attn, b_attn):
    wlt = W_lin.T
    bl = b_lin.reshape(1, OPH)
    wa = W_attn[0]
    wab = jnp.zeros((OPH, 8), jnp.float32)
    wab = wab.at[:, 0].set(wa[:OPH]).at[:, 1].set(wa[OPH:])
    bab = jnp.zeros((1, 8), jnp.float32).at[0, 1].set(b_attn[0])

    h, ab = _pre(x, wlt, bl, wab, bab)
    alpha = ab[:, 0]
    beta = ab[:, 1]

    pad = E_PAD - E
    s_p = jnp.pad(edge_index[0], (0, pad)).reshape(NW, NCHUNK, CHUNK)
    r_p = jnp.pad(edge_index[1], (0, pad)).reshape(NW, NCHUNK, CHUNK)
    z32 = jnp.zeros((N, OPH), jnp.float32)
    z1 = jnp.zeros((N,), jnp.float32)

    outp, denp = _sc_edge_pass(h, alpha, beta, s_p, r_p, z32, z1)
    return _post(outp, denp)


# speedup vs baseline: 36.5224x; 1.1987x over previous
import functools

import jax
import jax.numpy as jnp
from jax import lax
from jax.experimental import pallas as pl
from jax.experimental.pallas import tpu as pltpu
from jax.experimental.pallas import tpu_sc as plsc

N = 10000
E = 320000
IN_DIM = 128
OPH = 32

NC = 2
NS = 16
NW = NC * NS
CHUNK = 128
NCHUNK = 80
EPW = NCHUNK * CHUNK
E_PAD = NW * EPW


def _pre_body(x_ref, wlt_ref, bl_ref, wab_ref, bab_ref, h_ref, ab_ref):
    h = jnp.dot(x_ref[...], wlt_ref[...], preferred_element_type=jnp.float32)
    h = h + bl_ref[...]
    h_ref[...] = h
    ab_ref[...] = (
        jnp.dot(h, wab_ref[...], preferred_element_type=jnp.float32)
        + bab_ref[...]
    )


def _pre(x, wlt, bl, wab, bab):
    return pl.pallas_call(
        _pre_body,
        out_shape=(
            jax.ShapeDtypeStruct((N, OPH), jnp.float32),
            jax.ShapeDtypeStruct((N, 8), jnp.float32),
        ),
    )(x, wlt, bl, wab, bab)


def _sc_body(h_hbm, alpha_hbm, beta_hbm, s_hbm, r_hbm, z32_hbm, z1_hbm,
             outp_hbm, denp_hbm,
             alpha_t, beta_t, sbuf, rbuf, rows, scat, pfull, h_sh, out_acc,
             den_acc, gsem, ssem, dsem):
    cid = lax.axis_index("c")
    sid = lax.axis_index("s")
    wid = sid * NC + cid

    if True:
        pltpu.sync_copy(alpha_hbm, alpha_t)
        pltpu.sync_copy(beta_hbm, beta_t)
        pltpu.sync_copy(s_hbm.at[wid], sbuf)
        pltpu.sync_copy(r_hbm.at[wid], rbuf)

        @pl.when(sid == 0)
        def _():
            pltpu.sync_copy(z32_hbm, out_acc)
            pltpu.sync_copy(z1_hbm, den_acc)

        @pl.when(sid == 1)
        def _():
            pltpu.sync_copy(h_hbm, h_sh)

        ebase = wid * EPW
        lane = lax.iota(jnp.int32, 16)

        def _pchunk(c, carry):
            for j in range(CHUNK // 16):
                sv = sbuf[c, pl.ds(j * 16, 16)]
                rv = rbuf[c, pl.ds(j * 16, 16)]
                a = plsc.load_gather(alpha_t, [sv])
                b = plsc.load_gather(beta_t, [rv])
                l = a + b
                l = jnp.where(l >= 0.0, l, l * jnp.float32(0.01))
                p = jnp.exp(l)
                gid = ebase + c * CHUNK + j * 16 + lane
                p = jnp.where(gid < E, p, jnp.float32(0.0))
                pfull[c, pl.ds(j * 16, 16)] = p
            return carry

        lax.fori_loop(0, NCHUNK, _pchunk, 0)
        plsc.subcore_barrier()

        def _gstart(c, buf):
            pltpu.async_copy(h_sh.at[sbuf.at[c]], rows.at[buf], gsem.at[buf])

        def _gwait(c, buf):
            pltpu.make_async_copy(
                h_sh.at[sbuf.at[c]], rows.at[buf], gsem.at[buf]).wait()

        def _sstart(c, buf):
            pltpu.async_copy(scat.at[buf], out_acc.at[rbuf.at[c]],
                             ssem.at[buf], add=True)
            pltpu.async_copy(pfull.at[c], den_acc.at[rbuf.at[c]],
                             dsem.at[buf], add=True)

        def _swait(c, buf):
            pltpu.make_async_copy(scat.at[buf], out_acc.at[rbuf.at[c]],
                                  ssem.at[buf]).wait()
            pltpu.make_async_copy(pfull.at[c], den_acc.at[rbuf.at[c]],
                                  dsem.at[buf]).wait()

        _gstart(0, 0)
        _gstart(1, 1)

        def _process(c, buf):
            _gwait(c, buf)

            @pl.when(c >= 2)
            def _():
                _swait(c - 2, buf)

            cvec = jnp.broadcast_to(c, (16,)).astype(jnp.int32)
            for e in range(CHUNK):
                evec = jnp.full((16,), e, jnp.int32)
                pe = plsc.load_gather(pfull, [cvec, evec])
                scat[buf, e, pl.ds(0, 16)] = rows[buf, e, pl.ds(0, 16)] * pe
                scat[buf, e, pl.ds(16, 16)] = rows[buf, e, pl.ds(16, 16)] * pe
            _sstart(c, buf)

            @pl.when(c + 2 < NCHUNK)
            def _():
                _gstart(c + 2, buf)

        def _pair(t, carry):
            _process(2 * t, 0)
            _process(2 * t + 1, 1)
            return carry

        lax.fori_loop(0, NCHUNK // 2, _pair, 0)
        _swait(NCHUNK - 2, 0)
        _swait(NCHUNK - 1, 1)
        plsc.subcore_barrier()

        @pl.when(sid == 0)
        def _():
            pltpu.sync_copy(out_acc, outp_hbm.at[cid])
            pltpu.sync_copy(den_acc, denp_hbm.at[cid, 0])

def _sc_edge_pass(h, alpha, beta, s_p, r_p, z32, z1):
    mesh = plsc.VectorSubcoreMesh(core_axis_name="c", subcore_axis_name="s")
    return pl.kernel(
        _sc_body,
        out_type=(
            jax.ShapeDtypeStruct((NC, N, OPH), jnp.float32),
            jax.ShapeDtypeStruct((NC, 1, N), jnp.float32),
        ),
        mesh=mesh,
        compiler_params=pltpu.CompilerParams(
            needs_layout_passes=False, use_tc_tiling_on_sc=False),
        scratch_types=[
            pltpu.VMEM((N,), jnp.float32),
            pltpu.VMEM((N,), jnp.float32),
            pltpu.VMEM((NCHUNK, CHUNK), jnp.int32),
            pltpu.VMEM((NCHUNK, CHUNK), jnp.int32),
            pltpu.VMEM((2, CHUNK, OPH), jnp.float32),
            pltpu.VMEM((2, CHUNK, OPH), jnp.float32),
            pltpu.VMEM((NCHUNK, CHUNK), jnp.float32),
            pltpu.VMEM_SHARED((N, OPH), jnp.float32),
            pltpu.VMEM_SHARED((N, OPH), jnp.float32),
            pltpu.VMEM_SHARED((N,), jnp.float32),
            pltpu.SemaphoreType.DMA((2,)),
            pltpu.SemaphoreType.DMA((2,)),
            pltpu.SemaphoreType.DMA((2,)),
        ],
    )(h, alpha, beta, s_p, r_p, z32, z1)


def _post_body(op_ref, dp_ref, o_ref):
    num = op_ref[0] + op_ref[1]
    den = dp_ref[0, 0] + dp_ref[1, 0]
    den = den[:, None]
    o_ref[...] = jnp.where(den > 0.0, num / den, jnp.float32(0.0))


def _post(outp, denp):
    return pl.pallas_call(
        _post_body,
        out_shape=jax.ShapeDtypeStruct((N, OPH), jnp.float32),
    )(outp, denp)


@jax.jit
def kernel(x, edge_index, W_lin, b_lin, W_attn, b_attn):
    wlt = W_lin.T
    bl = b_lin.reshape(1, OPH)
    wa = W_attn[0]
    wab = jnp.zeros((OPH, 8), jnp.float32)
    wab = wab.at[:, 0].set(wa[:OPH]).at[:, 1].set(wa[OPH:])
    bab = jnp.zeros((1, 8), jnp.float32).at[0, 1].set(b_attn[0])

    h, ab = _pre(x, wlt, bl, wab, bab)
    alpha = ab[:, 0]
    beta = ab[:, 1]

    pad = E_PAD - E
    s_p = jnp.pad(edge_index[0], (0, pad)).reshape(NW, NCHUNK, CHUNK)
    r_p = jnp.pad(edge_index[1], (0, pad)).reshape(NW, NCHUNK, CHUNK)
    z32 = jnp.zeros((N, OPH), jnp.float32)
    z1 = jnp.zeros((N,), jnp.float32)

    outp, denp = _sc_edge_pass(h, alpha, beta, s_p, r_p, z32, z1)
    return _post(outp, denp)
